# baseline (speedup 1.0000x reference)
import functools

import jax
import jax.numpy as jnp
from jax import lax
from jax.experimental import pallas as pl
from jax.experimental.pallas import tpu as pltpu
import jax.experimental.pallas.tpu_sc as plsc

NC = 2
NS = 16
NW = NC * NS
CH = 64
NSLOT = 3


def _pad_slab(slab, start, total, value):
  vec = jnp.full((16,), value, jnp.int32)
  for i in range((total - start) // 16):
    slab[pl.ds(start + i * 16, 16)] = vec


def _hist_kernel(n, n_pad, e, n_chunks):
  mesh = plsc.VectorSubcoreMesh(
      core_axis_name="c", subcore_axis_name="s", num_cores=NC, num_subcores=NS)
  rows_per_tile = n_pad // NS
  et0 = e // NW
  et = n_chunks * CH
  assert et0 % 16 == 0 and et % 16 == 0

  @functools.partial(
      pl.kernel,
      out_type=(
          jax.ShapeDtypeStruct((n_pad,), jnp.float32),
          jax.ShapeDtypeStruct((n_pad,), jnp.float32),
      ),
      mesh=mesh,
      scratch_types=[
          pltpu.VMEM_SHARED((n_pad,), jnp.float32),
          pltpu.VMEM((et,), jnp.int32),
          pltpu.VMEM((CH,), jnp.float32),
          pltpu.VMEM((rows_per_tile,), jnp.float32),
      ],
  )
  def hist(edge_hbm, degA_hbm, degB_hbm, deg_sp, dst_v, ones_v, zeros_v):
    c = lax.axis_index("c")
    s = lax.axis_index("s")
    g = c * NS + s

    zvec = jnp.zeros((16,), jnp.float32)
    ovec = jnp.ones((16,), jnp.float32)
    for i in range(rows_per_tile // 16):
      zeros_v[pl.ds(i * 16, 16)] = zvec
    for i in range(CH // 16):
      ones_v[pl.ds(i * 16, 16)] = ovec

    base = s * rows_per_tile
    pltpu.sync_copy(zeros_v, deg_sp.at[pl.ds(base, rows_per_tile)])
    pltpu.sync_copy(edge_hbm.at[pl.ds(e + g * et0, et0)],
                    dst_v.at[pl.ds(0, et0)])
    _pad_slab(dst_v, et0, et, n)
    plsc.subcore_barrier()

    def body(j):
      pltpu.sync_copy(ones_v, deg_sp.at[dst_v.at[pl.ds(j * CH, CH)]],
                      add=True)
    pl.loop(0, n_chunks)(body)

    plsc.subcore_barrier()

    @pl.when(c == 0)
    def _():
      pltpu.sync_copy(deg_sp.at[pl.ds(base, rows_per_tile)],
                      degA_hbm.at[pl.ds(base, rows_per_tile)])

    @pl.when(c == 1)
    def _():
      pltpu.sync_copy(deg_sp.at[pl.ds(base, rows_per_tile)],
                      degB_hbm.at[pl.ds(base, rows_per_tile)])

  return hist


def _edge_kernel(n, n_pad, d, e, n_chunks):
  mesh = plsc.VectorSubcoreMesh(
      core_axis_name="c", subcore_axis_name="s", num_cores=NC, num_subcores=NS)
  rows_per_tile = n_pad // NS
  assert rows_per_tile % CH == 0
  n_zero = rows_per_tile // CH
  assert n_chunks % NSLOT == 0
  et0 = e // NW
  et = n_chunks * CH

  @functools.partial(
      pl.kernel,
      out_type=(
          jax.ShapeDtypeStruct((n_pad, d), jnp.float32),
          jax.ShapeDtypeStruct((n_pad, d), jnp.float32),
      ),
      mesh=mesh,
      scratch_types=[
          pltpu.VMEM_SHARED((n_pad, d), jnp.float32),
          pltpu.VMEM((et,), jnp.int32),
          pltpu.VMEM((et,), jnp.int32),
          [pltpu.VMEM((CH, d), jnp.float32) for _ in range(NSLOT)],
          [pltpu.SemaphoreType.DMA for _ in range(NSLOT)],
          [pltpu.SemaphoreType.DMA for _ in range(NSLOT)],
      ],
  )
  def edge(h2_hbm, edge_hbm, pA_hbm, pB_hbm,
           acc_sp, src_v, dst_v, bufs, gsems, ssems):
    c = lax.axis_index("c")
    s = lax.axis_index("s")
    g = c * NS + s

    base = s * rows_per_tile

    @pl.when(c == 0)
    def _():
      pltpu.sync_copy(h2_hbm.at[pl.ds(base, rows_per_tile)],
                      acc_sp.at[pl.ds(base, rows_per_tile)])

    @pl.when(c == 1)
    def _():
      zvec = jnp.zeros((16,), jnp.float32)

      def zero_row(i):
        for jj in range(d // 16):
          bufs[0][i, pl.ds(jj * 16, 16)] = zvec
      pl.loop(0, CH)(zero_row)
      for r in range(n_zero):
        pltpu.sync_copy(bufs[0], acc_sp.at[pl.ds(base + r * CH, CH)])

    pltpu.sync_copy(edge_hbm.at[pl.ds(g * et0, et0)],
                    src_v.at[pl.ds(0, et0)])
    pltpu.sync_copy(edge_hbm.at[pl.ds(e + g * et0, et0)],
                    dst_v.at[pl.ds(0, et0)])
    _pad_slab(src_v, et0, et, 0)
    _pad_slab(dst_v, et0, et, n)
    plsc.subcore_barrier()

    def sidx(j):
      return src_v.at[pl.ds(j * CH, CH)]

    def didx(j):
      return dst_v.at[pl.ds(j * CH, CH)]

    for k in range(NSLOT):
      pltpu.async_copy(h2_hbm.at[sidx(k)], bufs[k], gsems[k])

    def body(i):
      j = i * NSLOT
      descs = []
      for k in range(NSLOT):
        pltpu.make_async_copy(h2_hbm.at[sidx(j + k)], bufs[k], gsems[k]).wait()
        descs.append(pltpu.async_copy(
            bufs[k], acc_sp.at[didx(j + k)], ssems[k], add=True))
      for k in range(NSLOT):
        @pl.when(j + k + NSLOT < n_chunks)
        def _(k=k):
          descs[k].wait()
          pltpu.async_copy(h2_hbm.at[sidx(j + k + NSLOT)], bufs[k], gsems[k])

    pl.loop(0, n_chunks // NSLOT)(body)

    for k in range(NSLOT):
      pltpu.make_async_copy(
          bufs[k], acc_sp.at[didx(n_chunks - NSLOT + k)], ssems[k]).wait()

    plsc.subcore_barrier()

    @pl.when(c == 0)
    def _():
      pltpu.sync_copy(acc_sp.at[pl.ds(base, rows_per_tile)],
                      pA_hbm.at[pl.ds(base, rows_per_tile)])

    @pl.when(c == 1)
    def _():
      pltpu.sync_copy(acc_sp.at[pl.ds(base, rows_per_tile)],
                      pB_hbm.at[pl.ds(base, rows_per_tile)])

  return edge


def _unpack_rowscalars(d_block, bs):
  rows = d_block.shape[0]
  j_iota = lax.broadcasted_iota(jnp.int32, (bs, rows), 1)
  rdiv = lax.broadcasted_iota(jnp.int32, (bs, rows), 0) // 128
  sel = (j_iota == rdiv).astype(jnp.float32)
  g = lax.dot_general(sel, d_block, (((1,), (0,)), ((), ())),
                      precision=lax.Precision.HIGHEST,
                      preferred_element_type=jnp.float32)
  r_iota = lax.broadcasted_iota(jnp.int32, (bs, 128), 0)
  c_iota = lax.broadcasted_iota(jnp.int32, (bs, 128), 1)
  lane = (c_iota == r_iota % 128).astype(jnp.float32)
  return jnp.sum(g * lane, axis=1, keepdims=True)


def _matmul_body(x_ref, w_ref, dinv_ref, o_ref):
  bs = o_ref.shape[0]
  dinv = _unpack_rowscalars(dinv_ref[...], bs)
  h = lax.dot_general(x_ref[...], w_ref[...], (((1,), (1,)), ((), ())),
                      preferred_element_type=jnp.float32)
  o_ref[...] = h * dinv


def _combine_body(pA_ref, pB_ref, dinv_ref, b_ref, o_ref):
  bs = pA_ref.shape[0]
  dinv = _unpack_rowscalars(dinv_ref[...], bs)
  o_ref[...] = (pA_ref[...] + pB_ref[...]) * dinv + b_ref[...]


def kernel(x, edge_index, W, b):
  n, d_in = x.shape
  d_out = W.shape[0]
  e = edge_index.shape[1]

  n_pad = ((n + (NS * CH) - 1) // (NS * CH)) * (NS * CH)
  et0 = e // NW
  n_chunks = -(-et0 // CH)
  n_chunks = -(-n_chunks // NSLOT) * NSLOT
  assert n_chunks * CH * NW >= e

  ei_flat = edge_index.reshape(2 * e)
  degA, degB = _hist_kernel(n, n_pad, e, n_chunks)(ei_flat)
  dinv2d = lax.rsqrt(degA + degB + 1.0).reshape(n_pad // 128, 128)

  nb = 10
  bs2 = n_pad // nb
  h2 = pl.pallas_call(
      _matmul_body,
      grid=(nb,),
      in_specs=[
          pl.BlockSpec((bs2, d_in), lambda i: (i, 0)),
          pl.BlockSpec((d_out, d_in), lambda i: (0, 0)),
          pl.BlockSpec((bs2 // 128, 128), lambda i: (i, 0)),
      ],
      out_specs=pl.BlockSpec((bs2, d_out), lambda i: (i, 0)),
      out_shape=jax.ShapeDtypeStruct((n_pad, d_out), jnp.float32),
  )(x, W, dinv2d)

  pA, pB = _edge_kernel(n, n_pad, d_out, e, n_chunks)(h2, ei_flat)

  out = pl.pallas_call(
      _combine_body,
      grid=(nb,),
      in_specs=[
          pl.BlockSpec((bs2, d_out), lambda i: (i, 0)),
          pl.BlockSpec((bs2, d_out), lambda i: (i, 0)),
          pl.BlockSpec((bs2 // 128, 128), lambda i: (i, 0)),
          pl.BlockSpec((1, d_out), lambda i: (0, 0)),
      ],
      out_specs=pl.BlockSpec((bs2, d_out), lambda i: (i, 0)),
      out_shape=jax.ShapeDtypeStruct((n, d_out), jnp.float32),
  )(pA, pB, dinv2d, b.reshape(1, d_out))

  return out


# speedup vs baseline: 1.9771x; 1.9771x over previous
import functools

import jax
import jax.numpy as jnp
from jax import lax
from jax.experimental import pallas as pl
from jax.experimental.pallas import tpu as pltpu
import jax.experimental.pallas.tpu_sc as plsc

NC = 2
NS = 16
NW = NC * NS
CH = 64
NSLOT = 3


def _hist_kernel(n_pad, n_chunks):
  mesh = plsc.VectorSubcoreMesh(
      core_axis_name="c", subcore_axis_name="s", num_cores=NC, num_subcores=NS)
  rows_per_tile = n_pad // NS

  @functools.partial(
      pl.kernel,
      out_type=(
          jax.ShapeDtypeStruct((n_pad,), jnp.float32),
          jax.ShapeDtypeStruct((n_pad,), jnp.float32),
      ),
      mesh=mesh,
      scratch_types=[
          pltpu.VMEM_SHARED((n_pad,), jnp.float32),
          pltpu.VMEM((n_chunks, CH), jnp.int32),
          pltpu.VMEM((CH,), jnp.float32),
          pltpu.VMEM((rows_per_tile,), jnp.float32),
      ],
  )
  def hist(dst_hbm, degA_hbm, degB_hbm, deg_sp, dst_v, ones_v, zeros_v):
    c = lax.axis_index("c")
    s = lax.axis_index("s")
    g = c * NS + s

    zvec = jnp.zeros((16,), jnp.float32)
    ovec = jnp.ones((16,), jnp.float32)
    for i in range(rows_per_tile // 16):
      zeros_v[pl.ds(i * 16, 16)] = zvec
    for i in range(CH // 16):
      ones_v[pl.ds(i * 16, 16)] = ovec

    base = s * rows_per_tile
    pltpu.sync_copy(zeros_v, deg_sp.at[pl.ds(base, rows_per_tile)])
    pltpu.sync_copy(dst_hbm.at[g], dst_v)
    plsc.subcore_barrier()

    def body(j):
      pltpu.sync_copy(ones_v, deg_sp.at[dst_v.at[j]], add=True)
    pl.loop(0, n_chunks)(body)

    plsc.subcore_barrier()

    @pl.when(c == 0)
    def _():
      pltpu.sync_copy(deg_sp.at[pl.ds(base, rows_per_tile)],
                      degA_hbm.at[pl.ds(base, rows_per_tile)])

    @pl.when(c == 1)
    def _():
      pltpu.sync_copy(deg_sp.at[pl.ds(base, rows_per_tile)],
                      degB_hbm.at[pl.ds(base, rows_per_tile)])

  return hist


def _edge_kernel(n, n_pad, d, n_chunks):
  mesh = plsc.VectorSubcoreMesh(
      core_axis_name="c", subcore_axis_name="s", num_cores=NC, num_subcores=NS)
  rows_per_tile = n_pad // NS
  assert rows_per_tile % CH == 0
  n_zero = rows_per_tile // CH
  assert n_chunks % NSLOT == 0
  et = n_chunks * CH

  @functools.partial(
      pl.kernel,
      out_type=(
          jax.ShapeDtypeStruct((n_pad, d), jnp.float32),
          jax.ShapeDtypeStruct((n_pad, d), jnp.float32),
      ),
      mesh=mesh,
      scratch_types=[
          pltpu.VMEM_SHARED((n_pad, d), jnp.float32),
          pltpu.VMEM((et,), jnp.int32),
          pltpu.VMEM((et,), jnp.int32),
          [pltpu.VMEM((CH, d), jnp.float32) for _ in range(NSLOT)],
          [pltpu.SemaphoreType.DMA for _ in range(NSLOT)],
          [pltpu.SemaphoreType.DMA for _ in range(NSLOT)],
      ],
  )
  def edge(h2_hbm, src_hbm, dst_hbm, pA_hbm, pB_hbm,
           acc_sp, src_v, dst_v, bufs, gsems, ssems):
    c = lax.axis_index("c")
    s = lax.axis_index("s")
    g = c * NS + s

    base = s * rows_per_tile

    @pl.when(c == 0)
    def _():
      pltpu.sync_copy(h2_hbm.at[pl.ds(base, rows_per_tile)],
                      acc_sp.at[pl.ds(base, rows_per_tile)])

    @pl.when(c == 1)
    def _():
      zvec = jnp.zeros((16,), jnp.float32)

      def zero_row(i):
        for jj in range(d // 16):
          bufs[0][i, pl.ds(jj * 16, 16)] = zvec
      pl.loop(0, CH)(zero_row)
      for r in range(n_zero):
        pltpu.sync_copy(bufs[0], acc_sp.at[pl.ds(base + r * CH, CH)])

    pltpu.sync_copy(src_hbm.at[g], src_v)
    pltpu.sync_copy(dst_hbm.at[g], dst_v)
    plsc.subcore_barrier()

    def sidx(j):
      return src_v.at[pl.ds(j * CH, CH)]

    def didx(j):
      return dst_v.at[pl.ds(j * CH, CH)]

    for k in range(NSLOT):
      pltpu.async_copy(h2_hbm.at[sidx(k)], bufs[k], gsems[k])

    def body(i):
      j = i * NSLOT
      descs = []
      for k in range(NSLOT):
        pltpu.make_async_copy(h2_hbm.at[sidx(j + k)], bufs[k], gsems[k]).wait()
        descs.append(pltpu.async_copy(
            bufs[k], acc_sp.at[didx(j + k)], ssems[k], add=True))
      for k in range(NSLOT):
        @pl.when(j + k + NSLOT < n_chunks)
        def _(k=k):
          descs[k].wait()
          pltpu.async_copy(h2_hbm.at[sidx(j + k + NSLOT)], bufs[k], gsems[k])

    pl.loop(0, n_chunks // NSLOT)(body)

    for k in range(NSLOT):
      pltpu.make_async_copy(
          bufs[k], acc_sp.at[didx(n_chunks - NSLOT + k)], ssems[k]).wait()

    plsc.subcore_barrier()

    @pl.when(c == 0)
    def _():
      pltpu.sync_copy(acc_sp.at[pl.ds(base, rows_per_tile)],
                      pA_hbm.at[pl.ds(base, rows_per_tile)])

    @pl.when(c == 1)
    def _():
      pltpu.sync_copy(acc_sp.at[pl.ds(base, rows_per_tile)],
                      pB_hbm.at[pl.ds(base, rows_per_tile)])

  return edge


def _matmul_body(x_ref, w_ref, o_ref):
  o_ref[...] = lax.dot_general(x_ref[...], w_ref[...],
                               (((1,), (1,)), ((), ())),
                               preferred_element_type=jnp.float32)


def _unpack_rowscalars(d_block, bs):
  rows = d_block.shape[0]
  j_iota = lax.broadcasted_iota(jnp.int32, (bs, rows), 1)
  rdiv = lax.broadcasted_iota(jnp.int32, (bs, rows), 0) // 128
  sel = (j_iota == rdiv).astype(jnp.float32)
  g = lax.dot_general(sel, d_block, (((1,), (0,)), ((), ())),
                      precision=lax.Precision.HIGHEST,
                      preferred_element_type=jnp.float32)
  r_iota = lax.broadcasted_iota(jnp.int32, (bs, 128), 0)
  c_iota = lax.broadcasted_iota(jnp.int32, (bs, 128), 1)
  lane = (c_iota == r_iota % 128).astype(jnp.float32)
  return jnp.sum(g * lane, axis=1, keepdims=True)


def _scale_body(h_ref, dinv_ref, o_ref):
  bs = h_ref.shape[0]
  dinv = _unpack_rowscalars(dinv_ref[...], bs)
  o_ref[...] = h_ref[...] * dinv


def _combine_body(pA_ref, pB_ref, dinv_ref, b_ref, o_ref):
  bs = pA_ref.shape[0]
  dinv = _unpack_rowscalars(dinv_ref[...], bs)
  o_ref[...] = (pA_ref[...] + pB_ref[...]) * dinv + b_ref[...]


def kernel(x, edge_index, W, b):
  n, d_in = x.shape
  d_out = W.shape[0]
  e = edge_index.shape[1]

  n_pad = ((n + (NS * CH) - 1) // (NS * CH)) * (NS * CH)
  et = e // NW
  n_chunks = -(-et // CH)
  n_chunks = -(-n_chunks // NSLOT) * NSLOT
  e_pad = n_chunks * CH * NW

  ei = edge_index.reshape(2 * e)
  src = ei[:e]
  dst = ei[e:]
  pad = e_pad - e
  pad_src = jnp.arange(pad, dtype=jnp.int32) % n
  pad_dst = n + (jnp.arange(pad, dtype=jnp.int32) % (n_pad - n))
  srcp = jnp.concatenate([src, pad_src]).reshape(NW, n_chunks * CH)
  dstp = jnp.concatenate([dst, pad_dst]).reshape(NW, n_chunks * CH)

  degA, degB = _hist_kernel(n_pad, n_chunks)(
      dstp.reshape(NW, n_chunks, CH))
  dinv2d = lax.rsqrt(degA + degB + 1.0).reshape(n_pad // 128, 128)

  nb = 10
  bs = n // nb
  h = pl.pallas_call(
      _matmul_body,
      grid=(nb,),
      in_specs=[
          pl.BlockSpec((bs, d_in), lambda i: (i, 0)),
          pl.BlockSpec((d_out, d_in), lambda i: (0, 0)),
      ],
      out_specs=pl.BlockSpec((bs, d_out), lambda i: (i, 0)),
      out_shape=jax.ShapeDtypeStruct((n, d_out), jnp.float32),
  )(x, W)

  bs2 = n_pad // nb
  h2 = pl.pallas_call(
      _scale_body,
      grid=(nb,),
      in_specs=[
          pl.BlockSpec((bs2, d_out), lambda i: (i, 0)),
          pl.BlockSpec((bs2 // 128, 128), lambda i: (i, 0)),
      ],
      out_specs=pl.BlockSpec((bs2, d_out), lambda i: (i, 0)),
      out_shape=jax.ShapeDtypeStruct((n_pad, d_out), jnp.float32),
  )(h, dinv2d)

  pA, pB = _edge_kernel(n, n_pad, d_out, n_chunks)(h2, srcp, dstp)

  out = pl.pallas_call(
      _combine_body,
      grid=(nb,),
      in_specs=[
          pl.BlockSpec((bs2, d_out), lambda i: (i, 0)),
          pl.BlockSpec((bs2, d_out), lambda i: (i, 0)),
          pl.BlockSpec((bs2 // 128, 128), lambda i: (i, 0)),
          pl.BlockSpec((1, d_out), lambda i: (0, 0)),
      ],
      out_specs=pl.BlockSpec((bs2, d_out), lambda i: (i, 0)),
      out_shape=jax.ShapeDtypeStruct((n, d_out), jnp.float32),
  )(pA, pB, dinv2d, b.reshape(1, d_out))

  return out
